# baseline (speedup 1.0000x reference)
import functools

import jax
import jax.numpy as jnp
from jax import lax
from jax.experimental import pallas as pl
from jax.experimental.pallas import tpu as pltpu
from jax.experimental.pallas import tpu_sc as plsc

_NUM_E = 1000000
_D = 32
_BATCH = 16384
_FIELDS = 26
_B = _BATCH * _FIELDS

_NC = 2
_NS = 16
_NW = _NC * _NS
_BPW = _B // _NW

_CHUNK = 128
_GROUP = 4
_ROWS = _CHUNK * _GROUP
_NG = _BPW // _ROWS
_NBUF = 2

assert _BPW % _ROWS == 0 and (_NG - _NBUF) % _NBUF == 0


def _tp_body(in_ref, out_ref):
    out_ref[...] = in_ref[...].T


_TCOL = 8192
_TGRID = -(-_NUM_E // _TCOL)

_tc_transpose_table = pl.pallas_call(
    _tp_body,
    grid=(_TGRID,),
    in_specs=[pl.BlockSpec((_D, _TCOL), lambda j: (0, j))],
    out_specs=pl.BlockSpec((_TCOL, _D), lambda j: (j, 0)),
    out_shape=jax.ShapeDtypeStruct((_NUM_E, _D), jnp.float32),
)

_OBLK = 128
_OW = _FIELDS * _D

_tc_transpose_out = pl.pallas_call(
    _tp_body,
    grid=(_BATCH // _OBLK,),
    in_specs=[pl.BlockSpec((_OBLK, _OW), lambda j: (j, 0))],
    out_specs=pl.BlockSpec((_OW, _OBLK), lambda j: (0, j)),
    out_shape=jax.ShapeDtypeStruct((_OW, _BATCH), jnp.float32),
)


def _gather_body(table_hbm, idx_hbm, out_hbm, idx_v, rows_v, gsems, ssems):
    wid = lax.axis_index("s") * _NC + lax.axis_index("c")
    base = wid * _BPW
    pltpu.sync_copy(idx_hbm.at[pl.ds(base, _BPW)], idx_v)

    def fire_gather(g, b):
        for j in range(_GROUP):
            pltpu.async_copy(
                table_hbm.at[idx_v.at[pl.ds(g * _ROWS + j * _CHUNK, _CHUNK)]],
                rows_v.at[b].at[pl.ds(j * _CHUNK, _CHUNK)],
                gsems.at[b])

    def wait_gather(g, b):
        for j in range(_GROUP):
            pltpu.make_async_copy(
                table_hbm.at[idx_v.at[pl.ds(g * _ROWS + j * _CHUNK, _CHUNK)]],
                rows_v.at[b].at[pl.ds(j * _CHUNK, _CHUNK)],
                gsems.at[b]).wait()

    def fire_store(g, b):
        pltpu.async_copy(
            rows_v.at[b], out_hbm.at[pl.ds(base + g * _ROWS, _ROWS)],
            ssems.at[b])

    def wait_store(g, b):
        pltpu.make_async_copy(
            rows_v.at[b], out_hbm.at[pl.ds(base + g * _ROWS, _ROWS)],
            ssems.at[b]).wait()

    for b in range(_NBUF):
        fire_gather(b, b)

    @pl.loop(0, _NG - _NBUF, step=_NBUF)
    def _main(g0):
        for b in range(_NBUF):
            wait_gather(g0 + b, b)
            fire_store(g0 + b, b)
        for b in range(_NBUF):
            wait_store(g0 + b, b)
            fire_gather(g0 + _NBUF + b, b)

    for b in range(_NBUF):
        g = _NG - _NBUF + b
        wait_gather(g, b)
        fire_store(g, b)
    for b in range(_NBUF):
        wait_store(_NG - _NBUF + b, b)


_mesh = plsc.VectorSubcoreMesh(core_axis_name="c", subcore_axis_name="s")

_sc_gather = functools.partial(
    pl.kernel,
    out_type=jax.ShapeDtypeStruct((_B, _D), jnp.float32),
    mesh=_mesh,
    scratch_types=[
        pltpu.VMEM((_BPW,), jnp.int32),
        pltpu.VMEM((_NBUF, _ROWS, _D), jnp.float32),
        pltpu.SemaphoreType.DMA((_NBUF,)),
        pltpu.SemaphoreType.DMA((_NBUF,)),
    ],
    compiler_params=pltpu.CompilerParams(use_tc_tiling_on_sc=False),
)(_gather_body)


def kernel(ents, table):
    tbl_rm = _tc_transpose_table(table.T)
    idx = ents.reshape(-1).astype(jnp.int32)
    y = _sc_gather(tbl_rm, idx)
    yt = _tc_transpose_out(y.reshape(_BATCH, _OW))
    return yt.reshape(_FIELDS, _D, _BATCH).transpose(2, 0, 1)


# speedup vs baseline: 1.4322x; 1.4322x over previous
import functools

import jax
import jax.numpy as jnp
from jax import lax
from jax.experimental import pallas as pl
from jax.experimental.pallas import tpu as pltpu
from jax.experimental.pallas import tpu_sc as plsc

_NUM_E = 1000000
_D = 32
_BATCH = 16384
_FIELDS = 26
_B = _BATCH * _FIELDS

_NC = 2
_NS = 16
_NW = _NC * _NS
_BPW = _B // _NW

_CHUNK = 128
_GROUP = 4
_ROWS = _CHUNK * _GROUP
_NG = _BPW // _ROWS
_NBUF = 2

assert _BPW % _ROWS == 0 and (_NG - _NBUF) % _NBUF == 0


_PANEL = 8192
_QC = _PANEL // 4
_TGRID = -(-_NUM_E // _PANEL)
_SROWS = _TGRID * _QC


def _tt_body(in0, in1, in2, in3, out_ref):
    out_ref[...] = jnp.concatenate(
        [in0[...].T, in1[...].T, in2[...].T, in3[...].T], axis=1)


_tc_transpose_table = pl.pallas_call(
    _tt_body,
    grid=(_TGRID,),
    in_specs=[
        pl.BlockSpec(
            (_D, _QC),
            (lambda j, c=c: (0, jnp.minimum(4 * j + c, _NUM_E // _QC))))
        for c in range(4)
    ],
    out_specs=pl.BlockSpec((_QC, 128), lambda j: (j, 0)),
    out_shape=jax.ShapeDtypeStruct((_SROWS, 128), jnp.float32),
)


_OBK = 2048


def _ot_body(in_ref, out_ref):
    out_ref[0] = jnp.concatenate(
        [in_ref[:, 32 * u:32 * u + 32].T for u in range(4)], axis=1)


_tc_transpose_out = pl.pallas_call(
    _ot_body,
    grid=(_FIELDS, _BATCH // _OBK),
    in_specs=[pl.BlockSpec((512, 128),
                           lambda f, k: (f * (_BATCH // _OBK) + k, 0))],
    out_specs=pl.BlockSpec((1, _D, _OBK), lambda f, k: (f, 0, k)),
    out_shape=jax.ShapeDtypeStruct((_FIELDS, _D, _BATCH), jnp.float32),
)


def _gather_body(table_hbm, idx_hbm, out_hbm, idx_v, rows_v, gsems, ssems):
    wid = lax.axis_index("s") * _NC + lax.axis_index("c")
    base = wid * _BPW
    pltpu.sync_copy(idx_hbm.at[pl.ds(base, _BPW)], idx_v)

    def fire_gather(g, b):
        for j in range(_GROUP):
            pltpu.async_copy(
                table_hbm.at[idx_v.at[pl.ds(g * _ROWS + j * _CHUNK, _CHUNK)]],
                rows_v.at[b].at[pl.ds(j * _CHUNK, _CHUNK)],
                gsems.at[b])

    def wait_gather(g, b):
        for j in range(_GROUP):
            pltpu.make_async_copy(
                table_hbm.at[idx_v.at[pl.ds(g * _ROWS + j * _CHUNK, _CHUNK)]],
                rows_v.at[b].at[pl.ds(j * _CHUNK, _CHUNK)],
                gsems.at[b]).wait()

    def fire_store(g, b):
        pltpu.async_copy(
            rows_v.at[b], out_hbm.at[pl.ds(base + g * _ROWS, _ROWS)],
            ssems.at[b])

    def wait_store(g, b):
        pltpu.make_async_copy(
            rows_v.at[b], out_hbm.at[pl.ds(base + g * _ROWS, _ROWS)],
            ssems.at[b]).wait()

    for b in range(_NBUF):
        fire_gather(b, b)

    @pl.loop(0, _NG - _NBUF, step=_NBUF)
    def _main(g0):
        for b in range(_NBUF):
            wait_gather(g0 + b, b)
            fire_store(g0 + b, b)
        for b in range(_NBUF):
            wait_store(g0 + b, b)
            fire_gather(g0 + _NBUF + b, b)

    for b in range(_NBUF):
        g = _NG - _NBUF + b
        wait_gather(g, b)
        fire_store(g, b)
    for b in range(_NBUF):
        wait_store(_NG - _NBUF + b, b)


_mesh = plsc.VectorSubcoreMesh(core_axis_name="c", subcore_axis_name="s")

_sc_gather = functools.partial(
    pl.kernel,
    out_type=jax.ShapeDtypeStruct((_B, _D), jnp.float32),
    mesh=_mesh,
    scratch_types=[
        pltpu.VMEM((_BPW,), jnp.int32),
        pltpu.VMEM((_NBUF, _ROWS, _D), jnp.float32),
        pltpu.SemaphoreType.DMA((_NBUF,)),
        pltpu.SemaphoreType.DMA((_NBUF,)),
    ],
    compiler_params=pltpu.CompilerParams(use_tc_tiling_on_sc=False),
)(_gather_body)


def kernel(ents, table):
    t_t = table.T
    scratch = _tc_transpose_table(t_t, t_t, t_t, t_t)
    i = ents.T.reshape(-1).astype(jnp.int32)
    i = (i.reshape(_B // _OBK, 4, _OBK // 4)
          .transpose(0, 2, 1).reshape(-1))
    m = i & (_PANEL - 1)
    idx = (i & ~(_PANEL - 1)) | ((m & (_QC - 1)) << 2) | (m >> 11)
    y = _sc_gather(scratch.reshape(_SROWS * 4, _D), idx)
    out3 = _tc_transpose_out(y.reshape(_B // 4, 128))
    return out3.transpose(2, 0, 1)


# speedup vs baseline: 1.6551x; 1.1556x over previous
import functools

import jax
import jax.numpy as jnp
from jax import lax
from jax.experimental import pallas as pl
from jax.experimental.pallas import tpu as pltpu
from jax.experimental.pallas import tpu_sc as plsc

_NUM_E = 1000000
_D = 32
_BATCH = 16384
_FIELDS = 26
_B = _BATCH * _FIELDS

_NC = 2
_NS = 16
_NW = _NC * _NS
_BPW = _B // _NW

_CHUNK = 128
_GROUP = 4
_ROWS = _CHUNK * _GROUP
_NG = _BPW // _ROWS
_NBUF = 2

assert _BPW % _ROWS == 0 and (_NG - _NBUF) % _NBUF == 0


_PANEL = 16384
_QC = _PANEL // 4
_QCB = _QC.bit_length() - 1
_TGRID = -(-_NUM_E // _PANEL)
_SROWS = _TGRID * _QC


def _tt_body(in0, in1, in2, in3, out_ref):
    out_ref[...] = jnp.concatenate(
        [in0[...].T, in1[...].T, in2[...].T, in3[...].T], axis=1)


_tc_transpose_table = pl.pallas_call(
    _tt_body,
    grid=(_TGRID,),
    in_specs=[
        pl.BlockSpec(
            (_D, _QC),
            (lambda j, c=c: (0, jnp.minimum(4 * j + c, _NUM_E // _QC))))
        for c in range(4)
    ],
    out_specs=pl.BlockSpec((_QC, 128), lambda j: (j, 0)),
    out_shape=jax.ShapeDtypeStruct((_SROWS, 128), jnp.float32),
)


_OBK = 8192


def _ot_body(in_ref, out_ref):
    out_ref[0] = jnp.concatenate(
        [in_ref[:, 32 * u:32 * u + 32].T for u in range(4)], axis=1)


_tc_transpose_out = pl.pallas_call(
    _ot_body,
    grid=(_FIELDS, _BATCH // _OBK),
    in_specs=[pl.BlockSpec((_OBK // 4, 128),
                           lambda f, k: (f * (_BATCH // _OBK) + k, 0))],
    out_specs=pl.BlockSpec((1, _D, _OBK), lambda f, k: (f, 0, k)),
    out_shape=jax.ShapeDtypeStruct((_FIELDS, _D, _BATCH), jnp.float32),
)


def _gather_body(table_hbm, idx_hbm, out_hbm, idx_v, rows_v, gsems, ssems):
    wid = lax.axis_index("s") * _NC + lax.axis_index("c")
    base = wid * _BPW
    pltpu.sync_copy(idx_hbm.at[pl.ds(base, _BPW)], idx_v)

    def fire_gather(g, b):
        for j in range(_GROUP):
            pltpu.async_copy(
                table_hbm.at[idx_v.at[pl.ds(g * _ROWS + j * _CHUNK, _CHUNK)]],
                rows_v.at[b].at[pl.ds(j * _CHUNK, _CHUNK)],
                gsems.at[b])

    def wait_gather(g, b):
        for j in range(_GROUP):
            pltpu.make_async_copy(
                table_hbm.at[idx_v.at[pl.ds(g * _ROWS + j * _CHUNK, _CHUNK)]],
                rows_v.at[b].at[pl.ds(j * _CHUNK, _CHUNK)],
                gsems.at[b]).wait()

    def fire_store(g, b):
        pltpu.async_copy(
            rows_v.at[b], out_hbm.at[pl.ds(base + g * _ROWS, _ROWS)],
            ssems.at[b])

    def wait_store(g, b):
        pltpu.make_async_copy(
            rows_v.at[b], out_hbm.at[pl.ds(base + g * _ROWS, _ROWS)],
            ssems.at[b]).wait()

    for b in range(_NBUF):
        fire_gather(b, b)

    @pl.loop(0, _NG - _NBUF, step=_NBUF)
    def _main(g0):
        for b in range(_NBUF):
            wait_gather(g0 + b, b)
            fire_store(g0 + b, b)
        for b in range(_NBUF):
            wait_store(g0 + b, b)
            fire_gather(g0 + _NBUF + b, b)

    for b in range(_NBUF):
        g = _NG - _NBUF + b
        wait_gather(g, b)
        fire_store(g, b)
    for b in range(_NBUF):
        wait_store(_NG - _NBUF + b, b)


_mesh = plsc.VectorSubcoreMesh(core_axis_name="c", subcore_axis_name="s")

_sc_gather = functools.partial(
    pl.kernel,
    out_type=jax.ShapeDtypeStruct((_B, _D), jnp.float32),
    mesh=_mesh,
    scratch_types=[
        pltpu.VMEM((_BPW,), jnp.int32),
        pltpu.VMEM((_NBUF, _ROWS, _D), jnp.float32),
        pltpu.SemaphoreType.DMA((_NBUF,)),
        pltpu.SemaphoreType.DMA((_NBUF,)),
    ],
    compiler_params=pltpu.CompilerParams(use_tc_tiling_on_sc=False),
)(_gather_body)


def kernel(ents, table):
    t_t = table.T
    scratch = _tc_transpose_table(t_t, t_t, t_t, t_t)
    i = ents.T.reshape(-1).astype(jnp.int32)
    i = (i.reshape(_B // _OBK, 4, _OBK // 4)
          .transpose(0, 2, 1).reshape(-1))
    m = i & (_PANEL - 1)
    idx = (i & ~(_PANEL - 1)) | ((m & (_QC - 1)) << 2) | (m >> _QCB)
    y = _sc_gather(scratch.reshape(_SROWS * 4, _D), idx)
    out3 = _tc_transpose_out(y.reshape(_B // 4, 128))
    return out3.transpose(2, 0, 1)


# speedup vs baseline: 1.6669x; 1.0071x over previous
import functools

import jax
import jax.numpy as jnp
from jax import lax
from jax.experimental import pallas as pl
from jax.experimental.pallas import tpu as pltpu
from jax.experimental.pallas import tpu_sc as plsc

_NUM_E = 1000000
_D = 32
_BATCH = 16384
_FIELDS = 26
_B = _BATCH * _FIELDS

_NC = 2
_NS = 16
_NW = _NC * _NS
_BPW = _B // _NW

_CHUNK = 128
_GROUP = 4
_ROWS = _CHUNK * _GROUP
_NG = _BPW // _ROWS
_NBUF = 2

assert _BPW % _ROWS == 0 and (_NG - _NBUF) % _NBUF == 0


_PANEL = 32768
_QC = _PANEL // 4
_QCB = _QC.bit_length() - 1
_TGRID = -(-_NUM_E // _PANEL)
_SROWS = _TGRID * _QC


def _tt_body(in0, in1, in2, in3, out_ref):
    out_ref[...] = jnp.concatenate(
        [in0[...].T, in1[...].T, in2[...].T, in3[...].T], axis=1)


_tc_transpose_table = pl.pallas_call(
    _tt_body,
    grid=(_TGRID,),
    in_specs=[
        pl.BlockSpec(
            (_D, _QC),
            (lambda j, c=c: (0, jnp.minimum(4 * j + c, _NUM_E // _QC))))
        for c in range(4)
    ],
    out_specs=pl.BlockSpec((_QC, 128), lambda j: (j, 0)),
    out_shape=jax.ShapeDtypeStruct((_SROWS, 128), jnp.float32),
)


_OBK = 16384


def _ot_body(in_ref, out_ref):
    out_ref[0] = jnp.concatenate(
        [in_ref[:, 32 * u:32 * u + 32].T for u in range(4)], axis=1)


_tc_transpose_out = pl.pallas_call(
    _ot_body,
    grid=(_FIELDS, _BATCH // _OBK),
    in_specs=[pl.BlockSpec((_OBK // 4, 128),
                           lambda f, k: (f * (_BATCH // _OBK) + k, 0))],
    out_specs=pl.BlockSpec((1, _D, _OBK), lambda f, k: (f, 0, k)),
    out_shape=jax.ShapeDtypeStruct((_FIELDS, _D, _BATCH), jnp.float32),
)


def _gather_body(table_hbm, idx_hbm, out_hbm, idx_v, rows_v, gsems, ssems):
    wid = lax.axis_index("s") * _NC + lax.axis_index("c")
    base = wid * _BPW
    pltpu.sync_copy(idx_hbm.at[pl.ds(base, _BPW)], idx_v)

    def fire_gather(g, b):
        for j in range(_GROUP):
            pltpu.async_copy(
                table_hbm.at[idx_v.at[pl.ds(g * _ROWS + j * _CHUNK, _CHUNK)]],
                rows_v.at[b].at[pl.ds(j * _CHUNK, _CHUNK)],
                gsems.at[b])

    def wait_gather(g, b):
        for j in range(_GROUP):
            pltpu.make_async_copy(
                table_hbm.at[idx_v.at[pl.ds(g * _ROWS + j * _CHUNK, _CHUNK)]],
                rows_v.at[b].at[pl.ds(j * _CHUNK, _CHUNK)],
                gsems.at[b]).wait()

    def fire_store(g, b):
        pltpu.async_copy(
            rows_v.at[b], out_hbm.at[pl.ds(base + g * _ROWS, _ROWS)],
            ssems.at[b])

    def wait_store(g, b):
        pltpu.make_async_copy(
            rows_v.at[b], out_hbm.at[pl.ds(base + g * _ROWS, _ROWS)],
            ssems.at[b]).wait()

    for b in range(_NBUF):
        fire_gather(b, b)

    @pl.loop(0, _NG - _NBUF, step=_NBUF)
    def _main(g0):
        for b in range(_NBUF):
            wait_gather(g0 + b, b)
            fire_store(g0 + b, b)
        for b in range(_NBUF):
            wait_store(g0 + b, b)
            fire_gather(g0 + _NBUF + b, b)

    for b in range(_NBUF):
        g = _NG - _NBUF + b
        wait_gather(g, b)
        fire_store(g, b)
    for b in range(_NBUF):
        wait_store(_NG - _NBUF + b, b)


_mesh = plsc.VectorSubcoreMesh(core_axis_name="c", subcore_axis_name="s")

_sc_gather = functools.partial(
    pl.kernel,
    out_type=jax.ShapeDtypeStruct((_B, _D), jnp.float32),
    mesh=_mesh,
    scratch_types=[
        pltpu.VMEM((_BPW,), jnp.int32),
        pltpu.VMEM((_NBUF, _ROWS, _D), jnp.float32),
        pltpu.SemaphoreType.DMA((_NBUF,)),
        pltpu.SemaphoreType.DMA((_NBUF,)),
    ],
    compiler_params=pltpu.CompilerParams(use_tc_tiling_on_sc=False),
)(_gather_body)


def kernel(ents, table):
    t_t = table.T
    scratch = _tc_transpose_table(t_t, t_t, t_t, t_t)
    i = ents.T.reshape(-1).astype(jnp.int32)
    i = (i.reshape(_B // _OBK, 4, _OBK // 4)
          .transpose(0, 2, 1).reshape(-1))
    m = i & (_PANEL - 1)
    idx = (i & ~(_PANEL - 1)) | ((m & (_QC - 1)) << 2) | (m >> _QCB)
    y = _sc_gather(scratch.reshape(_SROWS * 4, _D), idx)
    out3 = _tc_transpose_out(y.reshape(_B // 4, 128))
    return out3.transpose(2, 0, 1)
